# baseline (speedup 1.0000x reference)
import functools

import jax
import jax.numpy as jnp
from jax import lax
from jax.experimental import pallas as pl
from jax.experimental.pallas import tpu as pltpu
import jax.experimental.pallas.tpu_sc as plsc

_R, _C = 4096, 1024
_BLK = 512
_G = _R // _BLK
_N = _R * _C

_NC, _NS, _L = 2, 16, 16
_NW = _NC * _NS
_PER_W = _N // _NW
_CHUNK = 16384
_NCHUNK = _PER_W // _CHUNK
_UNROLL = 4

_SQRT1_2 = 0.7071067811865476


def _erf(x):
    s = jnp.sign(x)
    a = jnp.abs(x)
    t = 1.0 / (1.0 + 0.3275911 * a)
    poly = t * (0.254829592 + t * (-0.284496736 + t * (1.421413741
               + t * (-1.453152027 + t * 1.061405429))))
    return s * (1.0 - poly * jnp.exp(-a * a))


def _sc_stats_body(p_hbm, t_hbm, out_hbm, pbuf, tbuf, obuf, sem):
    c = lax.axis_index("c")
    s = lax.axis_index("s")
    wid = s * _NC + c
    base = wid * _PER_W

    zeros = jnp.zeros((_L,), jnp.float32)
    ninf = jnp.full((_L,), -jnp.inf, jnp.float32)
    pinf = jnp.full((_L,), jnp.inf, jnp.float32)

    def chunk_step(k, carry):
        off = base + k * _CHUNK
        pltpu.sync_copy(p_hbm.at[pl.ds(off, _CHUNK)], pbuf)
        pltpu.sync_copy(t_hbm.at[pl.ds(off, _CHUNK)], tbuf)

        def vec_step(j, carry2):
            (t_sum, t_sumsq, t_max, p_sum, p_sumsq, p_max, p_min) = carry2
            jbase = j * (_L * _UNROLL)
            for u in range(_UNROLL):
                tv = tbuf[pl.ds(jbase + u * _L, _L)]
                pv = pbuf[pl.ds(jbase + u * _L, _L)]
                t_sum = t_sum + tv
                t_sumsq = t_sumsq + tv * tv
                t_max = jnp.maximum(t_max, tv)
                p_sum = p_sum + pv
                p_sumsq = p_sumsq + pv * pv
                p_max = jnp.maximum(p_max, pv)
                p_min = jnp.minimum(p_min, pv)
            return (t_sum, t_sumsq, t_max, p_sum, p_sumsq, p_max, p_min)

        return lax.fori_loop(0, _CHUNK // (_L * _UNROLL), vec_step, carry)

    init = (zeros, zeros, ninf, zeros, zeros, ninf, pinf)
    (t_sum, t_sumsq, t_max, p_sum, p_sumsq, p_max, p_min) = lax.fori_loop(
        0, _NCHUNK, chunk_step, init)

    for k, vec in enumerate(
            (t_sum, t_sumsq, t_max, p_sum, p_sumsq, p_max, p_min, zeros)):
        obuf[pl.ds(k * _L, _L)] = vec
    pltpu.sync_copy(obuf, out_hbm.at[pl.ds(wid * 8 * _L, 8 * _L)])


def _loss_body(stats_ref, p_ref, t_ref, out_ref, acc_ref):
    i = pl.program_id(0)
    n = jnp.float32(_N)
    s2 = stats_ref[...]
    t_sum = jnp.sum(s2[:, 0 * _L:1 * _L])
    t_sumsq = jnp.sum(s2[:, 1 * _L:2 * _L])
    t_maxv = jnp.max(s2[:, 2 * _L:3 * _L])
    p_sum = jnp.sum(s2[:, 3 * _L:4 * _L])
    p_sumsq = jnp.sum(s2[:, 4 * _L:5 * _L])
    p_max = jnp.max(s2[:, 5 * _L:6 * _L])
    p_min = jnp.min(s2[:, 6 * _L:7 * _L])

    t_mu = t_sum / n
    t_sd = jnp.sqrt((t_sumsq - n * t_mu * t_mu) / (n - 1.0))
    beta = 0.1 * t_maxv
    p_mu = p_sum / n
    p_sd = jnp.sqrt((p_sumsq - n * p_mu * p_mu) / (n - 1.0))

    inv_t = _SQRT1_2 / t_sd
    inv_p = _SQRT1_2 / p_sd

    p = p_ref[...]
    t = t_ref[...]

    diff = p - t
    absd = jnp.abs(diff)
    l1 = jnp.where(absd < beta, 0.5 * diff * diff / beta, absd - 0.5 * beta)

    cdf_t = 0.5 * (1.0 + _erf((t - t_mu) * inv_t))
    cdf_p = 0.5 * (1.0 + _erf((t - p_mu) * inv_p))
    lo = 0.5 * (1.0 + _erf((p_min - p_mu) * inv_p))
    hi = 0.5 * (1.0 + _erf((p_max - p_mu) * inv_p))
    cdf_p = jnp.clip(cdf_p, lo, hi)
    ks = jnp.abs(cdf_t - cdf_p)

    @pl.when(i == 0)
    def _init():
        acc_ref[0] = 0.0
        acc_ref[1] = -jnp.inf

    acc_ref[0] += jnp.sum(l1)
    acc_ref[1] = jnp.maximum(acc_ref[1], jnp.max(ks))

    @pl.when(i == _G - 1)
    def _done():
        out_ref[0] = 0.5 * (acc_ref[0] / n) + 0.5 * acc_ref[1]


@jax.jit
def _loss(predictions, targets):
    mesh = plsc.VectorSubcoreMesh(
        core_axis_name="c", subcore_axis_name="s", num_cores=_NC)
    stats = pl.kernel(
        _sc_stats_body,
        out_type=jax.ShapeDtypeStruct((_NW * 8 * _L,), jnp.float32),
        mesh=mesh,
        scratch_types=[
            pltpu.VMEM((_CHUNK,), jnp.float32),
            pltpu.VMEM((_CHUNK,), jnp.float32),
            pltpu.VMEM((8 * _L,), jnp.float32),
            pltpu.SemaphoreType.DMA,
        ],
    )(predictions.reshape(_N), targets.reshape(_N))
    stats = stats.reshape(_NW, 8 * _L)

    blk = pl.BlockSpec((_BLK, _C), lambda i: (i, 0))
    loss = pl.pallas_call(
        _loss_body,
        grid=(_G,),
        in_specs=[pl.BlockSpec((_NW, 8 * _L), lambda i: (0, 0)), blk, blk],
        out_specs=pl.BlockSpec(memory_space=pltpu.SMEM),
        out_shape=jax.ShapeDtypeStruct((1,), jnp.float32),
        scratch_shapes=[pltpu.SMEM((2,), jnp.float32)],
        compiler_params=pltpu.CompilerParams(
            dimension_semantics=("arbitrary",)),
    )(stats, predictions, targets)
    return loss[0]


def kernel(predictions, targets, write_idx=0):
    return _loss(predictions, targets)
